# baseline (speedup 1.0000x reference)
import functools

import jax
import jax.numpy as jnp
from jax.experimental import pallas as pl
from jax.experimental.pallas import tpu as pltpu
from jax.experimental.pallas import tpu_sc as plsc

NUM_HEADS = 16
NUM_GROUPS = 4
HPG = NUM_HEADS // NUM_GROUPS
NUM_VARS = 16
EMB_DIM = 256
BS = 2
SEQ = 2048
CH = 16
NW = 32


def _table_kernel(vid_ref, emb_ref, t_ref, idx_ref):
    n = pl.program_id(0)
    e = emb_ref[0]
    g = jax.lax.dot_general(
        e, e, (((1,), (1,)), ((), ())),
        preferred_element_type=jnp.float32,
        precision=jax.lax.Precision.HIGHEST,
    )
    v = vid_ref[...]
    iota_k = jax.lax.broadcasted_iota(jnp.int32, (NUM_VARS, SEQ), 0)
    onehot_k = (v == iota_k).astype(jnp.float32)
    t_ref[...] = jnp.dot(g, onehot_k, preferred_element_type=jnp.float32)
    idx_ref[...] = (v + n * NUM_VARS)[None]


def _sc_expand_body(idx_hbm, t_hbm, out_hbm, idx_v, rows_v, sem):
    cid = jax.lax.axis_index("c")
    sid = jax.lax.axis_index("s")
    wid = sid * 2 + cid
    n = jax.lax.rem(wid, NUM_HEADS)
    b = wid // NUM_HEADS
    base = b * (NUM_HEADS * SEQ) + n * SEQ

    pltpu.sync_copy(idx_hbm.at[n], idx_v)

    nit = SEQ // CH

    def gather(i, slot):
        return pltpu.make_async_copy(
            t_hbm.at[idx_v.at[pl.ds(i * CH, CH)]],
            rows_v.at[slot],
            sem.at[slot],
        )

    gather(0, 0).start()

    def body(i, slot):
        gather(i, slot).wait()
        nxt = 1 - slot

        @pl.when(i + 1 < nit)
        def _():
            gather(i + 1, nxt).start()

        pltpu.sync_copy(rows_v.at[slot], out_hbm.at[pl.ds(base + i * CH, CH)])
        return nxt

    jax.lax.fori_loop(0, nit, body, 0, unroll=2)


def kernel(query, key, query_id, kv_id, emb):
    del query, key, kv_id
    vid = query_id[0:1, :]
    emb_t = jnp.swapaxes(emb, 0, 1)

    t_tab, idx3 = pl.pallas_call(
        _table_kernel,
        grid=(NUM_HEADS,),
        in_specs=[
            pl.BlockSpec((1, SEQ), lambda n: (0, 0)),
            pl.BlockSpec((1, NUM_VARS, EMB_DIM), lambda n: (n, 0, 0)),
        ],
        out_specs=[
            pl.BlockSpec((NUM_VARS, SEQ), lambda n: (n, 0)),
            pl.BlockSpec((1, 1, SEQ), lambda n: (n, 0, 0)),
        ],
        out_shape=[
            jax.ShapeDtypeStruct((NUM_HEADS * NUM_VARS, SEQ), jnp.float32),
            jax.ShapeDtypeStruct((NUM_HEADS, 1, SEQ), jnp.int32),
        ],
    )(vid, emb_t)
    idx = idx3.reshape(NUM_HEADS, SEQ)

    mesh = plsc.VectorSubcoreMesh(core_axis_name="c", subcore_axis_name="s")
    sc_expand = functools.partial(
        pl.kernel,
        out_type=jax.ShapeDtypeStruct((BS * NUM_HEADS * SEQ, SEQ), jnp.float32),
        mesh=mesh,
        scratch_types=[
            pltpu.VMEM((SEQ,), jnp.int32),
            pltpu.VMEM((2, CH, SEQ), jnp.float32),
            pltpu.SemaphoreType.DMA((2,)),
        ],
    )(_sc_expand_body)

    out_flat = sc_expand(idx, t_tab)
    return out_flat.reshape(BS, NUM_HEADS, SEQ, SEQ).reshape(
        BS, NUM_GROUPS, HPG, SEQ, SEQ)


# speedup vs baseline: 2.7667x; 2.7667x over previous
import jax
import jax.numpy as jnp
from jax.experimental import pallas as pl

NUM_HEADS = 16
NUM_GROUPS = 4
HPG = NUM_HEADS // NUM_GROUPS
NUM_VARS = 16
EMB_DIM = 256
BS = 2
SEQ = 2048
TQ = 1024


def _bias_kernel(vid_ref, emb_ref, out_ref):
    qt = pl.program_id(1)
    e = emb_ref[0]
    g = jax.lax.dot_general(
        e, e, (((1,), (1,)), ((), ())),
        preferred_element_type=jnp.float32,
        precision=jax.lax.Precision.HIGHEST,
    )

    v = vid_ref[...]
    iota_k = jax.lax.broadcasted_iota(jnp.int32, (NUM_VARS, SEQ), 0)
    onehot_k = (v == iota_k).astype(jnp.float32)
    m = jnp.dot(g, onehot_k, preferred_element_type=jnp.float32)

    vq = vid_ref[0:1, pl.ds(qt * TQ, TQ)]
    iota_q = jax.lax.broadcasted_iota(jnp.int32, (NUM_VARS, TQ), 0)
    onehot_q = (vq == iota_q).astype(jnp.float32)
    tile = jax.lax.dot_general(
        onehot_q, m, (((0,), (0,)), ((), ())),
        preferred_element_type=jnp.float32,
    )

    out_ref[...] = jnp.broadcast_to(tile[None, None], (BS, 1, TQ, SEQ))


def kernel(query, key, query_id, kv_id, emb):
    del query, key, kv_id
    vid = query_id[0:1, :]
    emb_t = jnp.swapaxes(emb, 0, 1)
    nq = SEQ // TQ
    out = pl.pallas_call(
        _bias_kernel,
        grid=(NUM_HEADS, nq),
        in_specs=[
            pl.BlockSpec((1, SEQ), lambda n, qt: (0, 0)),
            pl.BlockSpec((1, NUM_VARS, EMB_DIM), lambda n, qt: (n, 0, 0)),
        ],
        out_specs=pl.BlockSpec((BS, 1, TQ, SEQ), lambda n, qt: (0, n, qt, 0)),
        out_shape=jax.ShapeDtypeStruct((BS, NUM_HEADS, SEQ, SEQ), jnp.float32),
    )(vid, emb_t)
    return out.reshape(BS, NUM_GROUPS, HPG, SEQ, SEQ)
